# baseline (speedup 1.0000x reference)
import functools
import math

import jax
import jax.numpy as jnp
from jax import lax
from jax.experimental import pallas as pl
from jax.experimental.pallas import tpu as pltpu
from jax.experimental.pallas import tpu_sc as plsc

_PAD = 0
_CONF = 0.9
_NC = 2
_NS = 16
_L = 16


def _sc_gather_body(V, R, logits_hbm, tok_hbm, out_hbm, idx_v, val_v, sem):
    c = lax.axis_index("c")
    s = lax.axis_index("s")
    wid = s * _NC + c
    rpw = R // _NS
    row_base = (wid % _NS) * rpw
    pltpu.sync_copy(tok_hbm.at[pl.ds(row_base, rpw)], idx_v)
    rows = row_base + lax.iota(jnp.int32, _L)
    is_le = jnp.where(wid < _NS, 1, 0)
    idx_v[...] = rows * V + idx_v[...] * is_le
    pltpu.async_copy(logits_hbm.at[idx_v], val_v, sem).wait()
    pltpu.sync_copy(val_v, out_hbm.at[pl.ds(wid * rpw, rpw)])


def _tc_rowsum_body(nblk, bw, V, x_ref, out_ref, acc_ref):
    i = pl.program_id(0)
    x = x_ref[...]
    col = i * bw + lax.broadcasted_iota(jnp.int32, x.shape, 1)
    part = jnp.sum(jnp.where(col < V, x, 0.0), axis=1, keepdims=True)

    @pl.when(i == 0)
    def _():
        acc_ref[...] = jnp.zeros_like(acc_ref)

    acc_ref[:, 0:1] += part

    @pl.when(i == nblk - 1)
    def _():
        out_ref[...] = acc_ref[:, 0:1]


def _tc_combine_body(V, tok_ref, le_ref, l0_ref, rowsum_ref, out_ref):
    d = (1.0 - _CONF) / (V - 2)
    a_const = (V - 2) * d * math.log(d) + _CONF * math.log(_CONF)
    rowsum = rowsum_ref[...]
    le = le_ref[...]
    l0 = l0_ref[...]
    tok = tok_ref[...]
    nonpad = (tok != _PAD).astype(jnp.float32)
    contrib = d * (rowsum - l0 - le) + _CONF * le
    n = jnp.sum(nonpad)
    tot = jnp.sum(contrib * nonpad)
    loss = (n * a_const - tot) / jnp.maximum(n, 1.0)
    out_ref[...] = jnp.full(out_ref.shape, loss)


def kernel(vocab_logits, expected_output_tokens, batch_idx):
    B, S, V = vocab_logits.shape
    R = B * S
    x2 = vocab_logits.reshape(R, V)
    x_flat = vocab_logits.reshape(R * V)
    tok_flat = expected_output_tokens.reshape(R).astype(jnp.int32)

    gathered = pl.kernel(
        functools.partial(_sc_gather_body, V, R),
        out_type=jax.ShapeDtypeStruct((2 * R,), jnp.float32),
        mesh=plsc.VectorSubcoreMesh(core_axis_name="c", subcore_axis_name="s"),
        scratch_types=[
            pltpu.VMEM((_L,), jnp.int32),
            pltpu.VMEM((_L,), jnp.float32),
            pltpu.SemaphoreType.DMA,
        ],
    )(x_flat, tok_flat)
    le2 = gathered[:R].reshape(R, 1)
    l02 = gathered[R:].reshape(R, 1)
    tok2 = tok_flat.reshape(R, 1)

    bw = 6272
    nblk = pl.cdiv(V, bw)
    rowsum = pl.pallas_call(
        functools.partial(_tc_rowsum_body, nblk, bw, V),
        grid=(nblk,),
        in_specs=[pl.BlockSpec((R, bw), lambda i: (0, i))],
        out_specs=pl.BlockSpec((R, 1), lambda i: (0, 0)),
        out_shape=jax.ShapeDtypeStruct((R, 1), jnp.float32),
        scratch_shapes=[pltpu.VMEM((R, 128), jnp.float32)],
    )(x2)
    out = pl.pallas_call(
        functools.partial(_tc_combine_body, V),
        in_specs=[
            pl.BlockSpec((R, 1), lambda: (0, 0)),
            pl.BlockSpec((R, 1), lambda: (0, 0)),
            pl.BlockSpec((R, 1), lambda: (0, 0)),
            pl.BlockSpec((R, 1), lambda: (0, 0)),
        ],
        out_specs=pl.BlockSpec((8, 128), lambda: (0, 0)),
        out_shape=jax.ShapeDtypeStruct((8, 128), jnp.float32),
    )(tok2, le2, l02, rowsum)
    return out[0, 0]


# speedup vs baseline: 5.2536x; 5.2536x over previous
import functools
import math

import jax
import jax.numpy as jnp
from jax import lax
from jax.experimental import pallas as pl
from jax.experimental.pallas import tpu as pltpu

_PAD = 0
_CONF = 0.9


def _body(nblk, bw, V, tok_ref, x_ref, out_ref, acc_ref, acc_le_ref, l0_ref):
    i = pl.program_id(0)
    rem = V - (nblk - 1) * bw
    x = x_ref[...]
    loc = tok_ref[...] - i * bw
    col = lax.broadcasted_iota(jnp.int32, x.shape, 1)
    le_part = jnp.sum(jnp.where(col == loc, x, 0.0), axis=1, keepdims=True)

    @pl.when(i == 0)
    def _():
        acc_ref[...] = jnp.zeros_like(acc_ref)
        acc_le_ref[...] = jnp.zeros_like(acc_le_ref)
        l0_ref[...] = x[:, 0:1]

    acc_le_ref[:, 0:1] += le_part

    @pl.when(i < nblk - 1)
    def _():
        acc_ref[:, 0:1] += jnp.sum(x, axis=1, keepdims=True)

    @pl.when(i == nblk - 1)
    def _():
        acc_ref[:, 0:1] += jnp.sum(x[:, :rem], axis=1, keepdims=True)
        d = (1.0 - _CONF) / (V - 2)
        a_const = (V - 2) * d * math.log(d) + _CONF * math.log(_CONF)
        rowsum = acc_ref[:, 0:1]
        le = acc_le_ref[:, 0:1]
        l0 = l0_ref[...]
        tok = tok_ref[...]
        nonpad = (tok != _PAD).astype(jnp.float32)
        contrib = d * (rowsum - l0 - le) + _CONF * le
        n = jnp.sum(nonpad)
        tot = jnp.sum(contrib * nonpad)
        loss = (n * a_const - tot) / jnp.maximum(n, 1.0)
        out_ref[...] = jnp.full(out_ref.shape, loss)


def kernel(vocab_logits, expected_output_tokens, batch_idx):
    B, S, V = vocab_logits.shape
    R = B * S
    x2 = vocab_logits.reshape(R, V)
    tok2 = expected_output_tokens.reshape(R, 1)
    bw = 6272
    nblk = pl.cdiv(V, bw)
    out = pl.pallas_call(
        functools.partial(_body, nblk, bw, V),
        grid=(nblk,),
        in_specs=[
            pl.BlockSpec((R, 1), lambda i: (0, 0)),
            pl.BlockSpec((R, bw), lambda i: (0, i)),
        ],
        out_specs=pl.BlockSpec((8, 128), lambda i: (0, 0)),
        out_shape=jax.ShapeDtypeStruct((8, 128), jnp.float32),
        scratch_shapes=[
            pltpu.VMEM((R, 128), jnp.float32),
            pltpu.VMEM((R, 128), jnp.float32),
            pltpu.VMEM((R, 1), jnp.float32),
        ],
    )(tok2, x2)
    return out[0, 0]


# speedup vs baseline: 5.5603x; 1.0584x over previous
import functools
import math

import jax
import jax.numpy as jnp
from jax import lax
from jax.experimental import pallas as pl
from jax.experimental.pallas import tpu as pltpu

_PAD = 0
_CONF = 0.9


def _body(nblk, bw, V, tok_ref, x_ref, out_ref, acc_ref, acc_le_ref, l0_ref):
    i = pl.program_id(0)
    rem = V - (nblk - 1) * bw
    x = x_ref[...]
    loc = tok_ref[...] - i * bw
    col = lax.broadcasted_iota(jnp.int32, x.shape, 1)
    le_part = jnp.sum(jnp.where(col == loc, x, 0.0), axis=1, keepdims=True)

    @pl.when(i == 0)
    def _():
        acc_ref[...] = jnp.zeros_like(acc_ref)
        acc_le_ref[...] = jnp.zeros_like(acc_le_ref)
        l0_ref[...] = x[:, 0:1]

    acc_le_ref[:, 0:1] += le_part

    @pl.when(i < nblk - 1)
    def _():
        acc_ref[:, 0:1] += jnp.sum(x, axis=1, keepdims=True)

    @pl.when(i == nblk - 1)
    def _():
        acc_ref[:, 0:1] += jnp.sum(x[:, :rem], axis=1, keepdims=True)
        d = (1.0 - _CONF) / (V - 2)
        a_const = (V - 2) * d * math.log(d) + _CONF * math.log(_CONF)
        rowsum = acc_ref[:, 0:1]
        le = acc_le_ref[:, 0:1]
        l0 = l0_ref[...]
        tok = tok_ref[...]
        nonpad = (tok != _PAD).astype(jnp.float32)
        contrib = d * (rowsum - l0 - le) + _CONF * le
        n = jnp.sum(nonpad)
        tot = jnp.sum(contrib * nonpad)
        loss = (n * a_const - tot) / jnp.maximum(n, 1.0)
        out_ref[...] = jnp.full(out_ref.shape, loss)


def kernel(vocab_logits, expected_output_tokens, batch_idx):
    B, S, V = vocab_logits.shape
    R = B * S
    x2 = vocab_logits.reshape(R, V)
    tok2 = expected_output_tokens.reshape(R, 1)
    bw = 12544
    nblk = pl.cdiv(V, bw)
    out = pl.pallas_call(
        functools.partial(_body, nblk, bw, V),
        grid=(nblk,),
        in_specs=[
            pl.BlockSpec((R, 1), lambda i: (0, 0)),
            pl.BlockSpec((R, bw), lambda i: (0, i)),
        ],
        out_specs=pl.BlockSpec((8, 128), lambda i: (0, 0)),
        out_shape=jax.ShapeDtypeStruct((8, 128), jnp.float32),
        scratch_shapes=[
            pltpu.VMEM((R, 128), jnp.float32),
            pltpu.VMEM((R, 128), jnp.float32),
            pltpu.VMEM((R, 1), jnp.float32),
        ],
    )(tok2, x2)
    return out[0, 0]
